# Initial kernel scaffold; baseline (speedup 1.0000x reference)
import functools
import jax
import jax.numpy as jnp
from jax import lax
from jax.experimental import pallas as pl
from jax.experimental.pallas import tpu as pltpu
from jax.experimental.pallas import tpu_sc as plsc

N = 10000
E = 320000
H = 128
NUM_RBF = 32
N_PAD = 10240

NC, NS = 2, 16
NW = NC * NS
EPT = E // NW
CH = 128
NFULL = EPT // CH
TAIL = EPT - NFULL * CH


def _silu(x):
    return x * jax.nn.sigmoid(x)


def _stage_a_body(h_ref, w1a_ref, w1b_ref, emb_ref, w1c_ref, b1_ref,
                  ha_ref, hb_ref, embc_ref):
    hv = h_ref[...]
    ha_ref[...] = jnp.dot(hv, w1a_ref[...], preferred_element_type=jnp.float32)
    hb_ref[...] = jnp.dot(hv, w1b_ref[...], preferred_element_type=jnp.float32)
    embc_ref[...] = (
        jnp.dot(emb_ref[...], w1c_ref[...], preferred_element_type=jnp.float32)
        + b1_ref[...])


def _stage_a(h, w1a, w1b, emb, w1c, b1):
    blk = 2000
    grid = N // blk
    return pl.pallas_call(
        _stage_a_body,
        grid=(grid,),
        in_specs=[
            pl.BlockSpec((blk, H), lambda i: (i, 0)),
            pl.BlockSpec((H, H), lambda i: (0, 0)),
            pl.BlockSpec((H, H), lambda i: (0, 0)),
            pl.BlockSpec((2, H), lambda i: (0, 0)),
            pl.BlockSpec((H, H), lambda i: (0, 0)),
            pl.BlockSpec((1, H), lambda i: (0, 0)),
        ],
        out_specs=[
            pl.BlockSpec((blk, H), lambda i: (i, 0)),
            pl.BlockSpec((blk, H), lambda i: (i, 0)),
            pl.BlockSpec((2, H), lambda i: (0, 0)),
        ],
        out_shape=[
            jax.ShapeDtypeStruct((N, H), jnp.float32),
            jax.ShapeDtypeStruct((N, H), jnp.float32),
            jax.ShapeDtypeStruct((2, H), jnp.float32),
        ],
    )(h, w1a, w1b, emb, w1c, b1)


def _stage_b_body(ha_hbm, hb_hbm, posp_hbm, negposp_hbm, src_hbm, dst_hbm,
                  preab_hbm, rel_hbm,
                  idx_s, idx_d, idx_s_t, idx_d_t, rows, relv, sem):
    wid = lax.axis_index("c") * NS + lax.axis_index("s")
    base0 = wid * EPT

    def chunk(i, carry):
        base = base0 + i * CH
        pltpu.sync_copy(src_hbm.at[pl.ds(base, CH)], idx_s)
        pltpu.sync_copy(dst_hbm.at[pl.ds(base, CH)], idx_d)
        pltpu.async_copy(ha_hbm.at[idx_s], rows, sem).wait()
        pltpu.async_copy(hb_hbm.at[idx_d], rows, sem, add=True).wait()
        pltpu.async_copy(posp_hbm.at[idx_s], relv, sem).wait()
        pltpu.async_copy(negposp_hbm.at[idx_d], relv, sem, add=True).wait()
        pltpu.sync_copy(rows, preab_hbm.at[pl.ds(base, CH)])
        pltpu.sync_copy(relv, rel_hbm.at[pl.ds(base, CH)])
        return carry

    lax.fori_loop(0, NFULL, chunk, 0)

    tb = base0 + NFULL * CH
    pltpu.sync_copy(src_hbm.at[pl.ds(tb, TAIL)], idx_s_t)
    pltpu.sync_copy(dst_hbm.at[pl.ds(tb, TAIL)], idx_d_t)
    pltpu.async_copy(ha_hbm.at[idx_s_t], rows.at[pl.ds(0, TAIL)], sem).wait()
    pltpu.async_copy(hb_hbm.at[idx_d_t], rows.at[pl.ds(0, TAIL)], sem,
                     add=True).wait()
    pltpu.async_copy(posp_hbm.at[idx_s_t], relv.at[pl.ds(0, TAIL)], sem).wait()
    pltpu.async_copy(negposp_hbm.at[idx_d_t], relv.at[pl.ds(0, TAIL)], sem,
                     add=True).wait()
    pltpu.sync_copy(rows.at[pl.ds(0, TAIL)], preab_hbm.at[pl.ds(tb, TAIL)])
    pltpu.sync_copy(relv.at[pl.ds(0, TAIL)], rel_hbm.at[pl.ds(tb, TAIL)])


def _stage_b(ha, hb, posp, negposp, src, dst):
    mesh = plsc.VectorSubcoreMesh(core_axis_name="c", subcore_axis_name="s",
                                  num_cores=NC, num_subcores=NS)
    f = pl.kernel(
        _stage_b_body,
        out_type=[
            jax.ShapeDtypeStruct((E, H), jnp.float32),
            jax.ShapeDtypeStruct((E, 16), jnp.float32),
        ],
        mesh=mesh,
        scratch_types=[
            pltpu.VMEM((CH,), jnp.int32),
            pltpu.VMEM((CH,), jnp.int32),
            pltpu.VMEM((TAIL,), jnp.int32),
            pltpu.VMEM((TAIL,), jnp.int32),
            pltpu.VMEM((CH, H), jnp.float32),
            pltpu.VMEM((CH, 16), jnp.float32),
            pltpu.SemaphoreType.DMA,
        ],
    )
    return f(ha, hb, posp, negposp, src, dst)


def _stage_c_body(preab_ref, rel_ref, et_ref, embc_ref, censg_ref, sg_ref,
                  w1d_ref, w2_ref, b2_ref, m_ref):
    relv = rel_ref[...]
    d2 = jnp.sum(relv * relv, axis=1, keepdims=True)
    dist = jnp.sqrt(d2)
    diff = dist * sg_ref[...] - censg_ref[...]
    radial = jnp.exp(-diff * diff)
    ec = embc_ref[...]
    et = et_ref[...]
    ek = ec[0:1, :] + et * (ec[1:2, :] - ec[0:1, :])
    pre = (preab_ref[...] + ek
           + jnp.dot(radial, w1d_ref[...], preferred_element_type=jnp.float32))
    x = _silu(pre)
    xm = jnp.dot(x, w2_ref[...], preferred_element_type=jnp.float32) + b2_ref[...]
    m_ref[...] = _silu(xm)


def _stage_c(preab, rel, etf, embc, censg, sg, w1d, w2, b2):
    blk = 2000
    grid = E // blk
    return pl.pallas_call(
        _stage_c_body,
        grid=(grid,),
        in_specs=[
            pl.BlockSpec((blk, H), lambda i: (i, 0)),
            pl.BlockSpec((blk, 16), lambda i: (i, 0)),
            pl.BlockSpec((blk, 1), lambda i: (i, 0)),
            pl.BlockSpec((2, H), lambda i: (0, 0)),
            pl.BlockSpec((1, NUM_RBF), lambda i: (0, 0)),
            pl.BlockSpec((1, NUM_RBF), lambda i: (0, 0)),
            pl.BlockSpec((NUM_RBF, H), lambda i: (0, 0)),
            pl.BlockSpec((H, H), lambda i: (0, 0)),
            pl.BlockSpec((1, H), lambda i: (0, 0)),
        ],
        out_specs=pl.BlockSpec((blk, H), lambda i: (i, 0)),
        out_shape=jax.ShapeDtypeStruct((E, H), jnp.float32),
    )(preab, rel, etf, embc, censg, sg, w1d, w2, b2)


def _stage_d_body(m_hbm, dst_hbm, zeros2_hbm, zeros1_hbm,
                  sums2_hbm, cnt2_hbm,
                  ssum, scnt, idx, idx_t, rows, ones, sem):
    cid = lax.axis_index("c")
    sid = lax.axis_index("s")
    rpt = N_PAD // NS
    rbase = sid * rpt

    pltpu.sync_copy(zeros2_hbm.at[pl.ds(rbase, rpt)], ssum.at[pl.ds(rbase, rpt)])
    pltpu.sync_copy(zeros1_hbm.at[pl.ds(rbase, rpt)], scnt.at[pl.ds(rbase, rpt)])

    def set_ones(i, carry):
        ones[pl.ds(i * 16, 16)] = jnp.full((16,), 1.0, jnp.float32)
        return carry

    lax.fori_loop(0, CH // 16, set_ones, 0)
    plsc.subcore_barrier()

    base0 = (cid * NS + sid) * EPT

    def chunk(i, carry):
        base = base0 + i * CH
        pltpu.sync_copy(dst_hbm.at[pl.ds(base, CH)], idx)
        pltpu.sync_copy(m_hbm.at[pl.ds(base, CH)], rows)
        pltpu.sync_copy(rows, ssum.at[idx], add=True)
        pltpu.sync_copy(ones, scnt.at[idx], add=True)
        return carry

    lax.fori_loop(0, NFULL, chunk, 0)

    tb = base0 + NFULL * CH
    pltpu.sync_copy(dst_hbm.at[pl.ds(tb, TAIL)], idx_t)
    pltpu.sync_copy(m_hbm.at[pl.ds(tb, TAIL)], rows.at[pl.ds(0, TAIL)])
    pltpu.sync_copy(rows.at[pl.ds(0, TAIL)], ssum.at[idx_t], add=True)
    pltpu.sync_copy(ones.at[pl.ds(0, TAIL)], scnt.at[idx_t], add=True)

    plsc.subcore_barrier()
    pltpu.sync_copy(ssum.at[pl.ds(rbase, rpt)],
                    sums2_hbm.at[cid, pl.ds(rbase, rpt)])
    pltpu.sync_copy(scnt.at[pl.ds(rbase, rpt)],
                    cnt2_hbm.at[cid, pl.ds(rbase, rpt)])


def _stage_d(m, dst, zeros2, zeros1):
    mesh = plsc.VectorSubcoreMesh(core_axis_name="c", subcore_axis_name="s",
                                  num_cores=NC, num_subcores=NS)
    f = pl.kernel(
        _stage_d_body,
        out_type=[
            jax.ShapeDtypeStruct((NC, N_PAD, H), jnp.float32),
            jax.ShapeDtypeStruct((NC, N_PAD), jnp.float32),
        ],
        mesh=mesh,
        scratch_types=[
            pltpu.VMEM_SHARED((N_PAD, H), jnp.float32),
            pltpu.VMEM_SHARED((N_PAD,), jnp.float32),
            pltpu.VMEM((CH,), jnp.int32),
            pltpu.VMEM((TAIL,), jnp.int32),
            pltpu.VMEM((CH, H), jnp.float32),
            pltpu.VMEM((CH,), jnp.float32),
            pltpu.SemaphoreType.DMA,
        ],
    )
    return f(m, dst, zeros2, zeros1)


def _stage_e_body(sums2_ref, cnt2_ref, h_ref, mask_ref,
                  u1a_ref, u1b_ref, u1v_ref, u2m_ref, u2v_ref,
                  g_ref, b_ref, out_ref):
    s = sums2_ref[0] + sums2_ref[1]
    c = cnt2_ref[0] + cnt2_ref[1]
    m_i = s / jnp.maximum(c, 1.0)
    hv = h_ref[...]
    u = _silu(jnp.dot(hv, u1a_ref[...], preferred_element_type=jnp.float32)
              + jnp.dot(m_i, u1b_ref[...], preferred_element_type=jnp.float32)
              + u1v_ref[...])
    upd = jnp.dot(u, u2m_ref[...], preferred_element_type=jnp.float32) + u2v_ref[...]
    y = hv + upd
    mu = jnp.mean(y, axis=1, keepdims=True)
    var = jnp.mean((y - mu) ** 2, axis=1, keepdims=True)
    yn = (y - mu) / jnp.sqrt(var + 1e-5) * g_ref[...] + b_ref[...]
    out_ref[...] = jnp.where(mask_ref[...] > 0.5, yn, hv)


def _stage_e(sums2, cnt2r, h_pad, mask, u1a, u1b, u1v, u2m, u2v, g, b):
    blk = 1280
    grid = N_PAD // blk
    return pl.pallas_call(
        _stage_e_body,
        grid=(grid,),
        in_specs=[
            pl.BlockSpec((NC, blk, H), lambda i: (0, i, 0)),
            pl.BlockSpec((NC, blk, 1), lambda i: (0, i, 0)),
            pl.BlockSpec((blk, H), lambda i: (i, 0)),
            pl.BlockSpec((blk, 1), lambda i: (i, 0)),
            pl.BlockSpec((H, H), lambda i: (0, 0)),
            pl.BlockSpec((H, H), lambda i: (0, 0)),
            pl.BlockSpec((1, H), lambda i: (0, 0)),
            pl.BlockSpec((H, H), lambda i: (0, 0)),
            pl.BlockSpec((1, H), lambda i: (0, 0)),
            pl.BlockSpec((1, H), lambda i: (0, 0)),
            pl.BlockSpec((1, H), lambda i: (0, 0)),
        ],
        out_specs=pl.BlockSpec((blk, H), lambda i: (i, 0)),
        out_shape=jax.ShapeDtypeStruct((N_PAD, H), jnp.float32),
    )(sums2, cnt2r, h_pad, mask, u1a, u1b, u1v, u2m, u2v, g, b)


@jax.jit
def kernel(h, pos, edge_index, edge_type, node_type, centers, emb,
           W1, b1, W2, b2, U1, u1, U2, u2, ln_g, ln_b):
    src = edge_index[0].astype(jnp.int32)
    dst = edge_index[1].astype(jnp.int32)

    w1a, w1b, w1c, w1d = W1[:H], W1[H:2 * H], W1[2 * H:3 * H], W1[3 * H:]
    step = centers[1] - centers[0]
    gamma = 1.0 / jnp.maximum(step * step, 1e-6)
    sg = jnp.sqrt(gamma)
    censg = (centers * sg).reshape(1, NUM_RBF)
    sg_arr = jnp.broadcast_to(sg, (1, NUM_RBF)).astype(jnp.float32)

    posp = jnp.zeros((N, 16), jnp.float32).at[:, :3].set(pos)
    negposp = -posp
    etf = edge_type.astype(jnp.float32).reshape(E, 1)

    ha, hb, embc = _stage_a(h, w1a, w1b, emb, w1c, b1.reshape(1, H))
    preab, rel = _stage_b(ha, hb, posp, negposp, src, dst)
    m = _stage_c(preab, rel, etf, embc, censg, sg_arr, w1d, W2,
                 b2.reshape(1, H))

    zeros2 = jnp.zeros((N_PAD, H), jnp.float32)
    zeros1 = jnp.zeros((N_PAD,), jnp.float32)
    sums2, cnt2 = _stage_d(m, dst, zeros2, zeros1)
    cnt2r = cnt2.reshape(NC, N_PAD, 1)

    h_pad = jnp.zeros((N_PAD, H), jnp.float32).at[:N].set(h)
    mask = jnp.zeros((N_PAD, 1), jnp.float32).at[:N, 0].set(
        (node_type == 1).astype(jnp.float32))

    u1a, u1b = U1[:H], U1[H:]
    out_pad = _stage_e(sums2, cnt2r, h_pad, mask, u1a, u1b,
                       u1.reshape(1, H), U2, u2.reshape(1, H),
                       ln_g.reshape(1, H), ln_b.reshape(1, H))
    return out_pad[:N]


# speedup vs baseline: 4.2671x; 4.2671x over previous
import jax
import jax.numpy as jnp
from jax import lax
from jax.experimental import pallas as pl
from jax.experimental.pallas import tpu as pltpu
from jax.experimental.pallas import tpu_sc as plsc

N = 10000
E = 320000
H = 128
NUM_RBF = 32
N_PAD = 10240

NC, NS = 2, 16
NW = NC * NS
EPT = E // NW
CH = 128
NFULL = EPT // CH
TAIL = EPT - NFULL * CH
POS_W = 4


def _silu(x):
    return x * jax.nn.sigmoid(x)


def _stage_a_body(h_ref, w1a_ref, w1b_ref, emb_ref, w1c_ref, b1_ref,
                  ha_ref, hb_ref, embc_ref):
    hv = h_ref[...]
    ha_ref[...] = jnp.dot(hv, w1a_ref[...], preferred_element_type=jnp.float32)
    hb_ref[...] = jnp.dot(hv, w1b_ref[...], preferred_element_type=jnp.float32)
    embc_ref[...] = (
        jnp.dot(emb_ref[...], w1c_ref[...], preferred_element_type=jnp.float32)
        + b1_ref[...])


def _stage_a(h, w1a, w1b, emb, w1c, b1):
    blk = 2000
    grid = N // blk
    return pl.pallas_call(
        _stage_a_body,
        grid=(grid,),
        in_specs=[
            pl.BlockSpec((blk, H), lambda i: (i, 0)),
            pl.BlockSpec((H, H), lambda i: (0, 0)),
            pl.BlockSpec((H, H), lambda i: (0, 0)),
            pl.BlockSpec((2, H), lambda i: (0, 0)),
            pl.BlockSpec((H, H), lambda i: (0, 0)),
            pl.BlockSpec((1, H), lambda i: (0, 0)),
        ],
        out_specs=[
            pl.BlockSpec((blk, H), lambda i: (i, 0)),
            pl.BlockSpec((blk, H), lambda i: (i, 0)),
            pl.BlockSpec((2, H), lambda i: (0, 0)),
        ],
        out_shape=[
            jax.ShapeDtypeStruct((N, H), jnp.float32),
            jax.ShapeDtypeStruct((N, H), jnp.float32),
            jax.ShapeDtypeStruct((2, H), jnp.float32),
        ],
    )(h, w1a, w1b, emb, w1c, b1)


def _stage_b_body(ha_hbm, hb_hbm, posflat_hbm, src_hbm, dst_hbm,
                  preab_hbm, d2_hbm,
                  idx_s, idx_d, idx_s_t, idx_d_t, rows, posv, d2buf, sem):
    wid = lax.axis_index("c") * NS + lax.axis_index("s")
    base0 = wid * EPT

    pltpu.sync_copy(posflat_hbm, posv)

    def dist2_group(g, is_tail):
        s16 = idx_s_t[...] if is_tail else idx_s[pl.ds(g * 16, 16)]
        d16 = idx_d_t[...] if is_tail else idx_d[pl.ds(g * 16, 16)]
        sb = s16 * POS_W
        db = d16 * POS_W
        acc = jnp.zeros((16,), jnp.float32)
        for k in range(3):
            a = plsc.load_gather(posv, [sb + k])
            b = plsc.load_gather(posv, [db + k])
            r = a - b
            acc = acc + r * r
        d2buf[pl.ds(g * 16, 16)] = acc

    def chunk(i, carry):
        base = base0 + i * CH
        pltpu.sync_copy(src_hbm.at[pl.ds(base, CH)], idx_s)
        pltpu.sync_copy(dst_hbm.at[pl.ds(base, CH)], idx_d)
        cp1 = pltpu.async_copy(ha_hbm.at[idx_s], rows, sem)
        cp1.wait()
        cp2 = pltpu.async_copy(hb_hbm.at[idx_d], rows, sem, add=True)
        for g in range(CH // 16):
            dist2_group(g, False)
        cp2.wait()
        pltpu.sync_copy(rows, preab_hbm.at[pl.ds(base, CH)])
        pltpu.sync_copy(d2buf, d2_hbm.at[pl.ds(base, CH)])
        return carry

    lax.fori_loop(0, NFULL, chunk, 0)

    tb = base0 + NFULL * CH
    pltpu.sync_copy(src_hbm.at[pl.ds(tb, TAIL)], idx_s_t)
    pltpu.sync_copy(dst_hbm.at[pl.ds(tb, TAIL)], idx_d_t)
    pltpu.async_copy(ha_hbm.at[idx_s_t], rows.at[pl.ds(0, TAIL)], sem).wait()
    pltpu.async_copy(hb_hbm.at[idx_d_t], rows.at[pl.ds(0, TAIL)], sem,
                     add=True).wait()
    dist2_group(0, True)
    pltpu.sync_copy(rows.at[pl.ds(0, TAIL)], preab_hbm.at[pl.ds(tb, TAIL)])
    pltpu.sync_copy(d2buf.at[pl.ds(0, TAIL)], d2_hbm.at[pl.ds(tb, TAIL)])


def _stage_b(ha, hb, posflat, src, dst):
    mesh = plsc.VectorSubcoreMesh(core_axis_name="c", subcore_axis_name="s",
                                  num_cores=NC, num_subcores=NS)
    f = pl.kernel(
        _stage_b_body,
        out_type=[
            jax.ShapeDtypeStruct((E, H), jnp.float32),
            jax.ShapeDtypeStruct((E,), jnp.float32),
        ],
        mesh=mesh,
        scratch_types=[
            pltpu.VMEM((CH,), jnp.int32),
            pltpu.VMEM((CH,), jnp.int32),
            pltpu.VMEM((TAIL,), jnp.int32),
            pltpu.VMEM((TAIL,), jnp.int32),
            pltpu.VMEM((CH, H), jnp.float32),
            pltpu.VMEM((N * POS_W,), jnp.float32),
            pltpu.VMEM((CH,), jnp.float32),
            pltpu.SemaphoreType.DMA,
        ],
        compiler_params=pltpu.CompilerParams(needs_layout_passes=False),
    )
    return f(ha, hb, posflat, src, dst)


def _stage_c_body(preab_ref, d2_ref, et_ref, dst_ref, embc_ref, censg_ref,
                  sg_ref, w1d_ref, w2_ref, b2_ref, m_ref, cnt_ref):
    i = pl.program_id(0)

    d2 = d2_ref[...]
    dist = jnp.sqrt(d2)
    diff = dist * sg_ref[...] - censg_ref[...]
    radial = jnp.exp(-diff * diff)
    ec = embc_ref[...]
    et = et_ref[...]
    ek = ec[0:1, :] + et * (ec[1:2, :] - ec[0:1, :])
    pre = (preab_ref[...] + ek
           + jnp.dot(radial, w1d_ref[...], preferred_element_type=jnp.float32))
    x = _silu(pre)
    xm = jnp.dot(x, w2_ref[...], preferred_element_type=jnp.float32) + b2_ref[...]
    m_ref[...] = _silu(xm)

    blk = dst_ref.shape[0]
    dstf = dst_ref[...]
    qf = jnp.floor(dstf * (1.0 / 128.0))
    rf = dstf - qf * 128.0
    lane = lax.broadcasted_iota(jnp.int32, (blk, 128), 1).astype(jnp.float32)
    oh_r = jnp.where(rf == lane, 1.0, 0.0)
    oh_q = jnp.where(qf == lane, 1.0, 0.0)
    part = lax.dot_general(oh_q, oh_r, (((0,), (0,)), ((), ())),
                           preferred_element_type=jnp.float32)

    @pl.when(i == 0)
    def _():
        cnt_ref[...] = jnp.zeros_like(cnt_ref)

    cnt_ref[...] += part


def _stage_c(preab, d2, etf, dstf, embc, censg, sg, w1d, w2, b2):
    blk = 2000
    grid = E // blk
    return pl.pallas_call(
        _stage_c_body,
        grid=(grid,),
        in_specs=[
            pl.BlockSpec((blk, H), lambda i: (i, 0)),
            pl.BlockSpec((blk, 1), lambda i: (i, 0)),
            pl.BlockSpec((blk, 1), lambda i: (i, 0)),
            pl.BlockSpec((blk, 1), lambda i: (i, 0)),
            pl.BlockSpec((2, H), lambda i: (0, 0)),
            pl.BlockSpec((1, NUM_RBF), lambda i: (0, 0)),
            pl.BlockSpec((1, NUM_RBF), lambda i: (0, 0)),
            pl.BlockSpec((NUM_RBF, H), lambda i: (0, 0)),
            pl.BlockSpec((H, H), lambda i: (0, 0)),
            pl.BlockSpec((1, H), lambda i: (0, 0)),
        ],
        out_specs=[
            pl.BlockSpec((blk, H), lambda i: (i, 0)),
            pl.BlockSpec((128, 128), lambda i: (0, 0)),
        ],
        out_shape=[
            jax.ShapeDtypeStruct((E, H), jnp.float32),
            jax.ShapeDtypeStruct((128, 128), jnp.float32),
        ],
    )(preab, d2, etf, dstf, embc, censg, sg, w1d, w2, b2)


def _stage_d_body(m_hbm, dst_hbm, zeros2_hbm, sums2_hbm,
                  ssum, idx, idx_t, rows, sem):
    cid = lax.axis_index("c")
    sid = lax.axis_index("s")
    rpt = N_PAD // NS
    rbase = sid * rpt

    pltpu.sync_copy(zeros2_hbm.at[pl.ds(rbase, rpt)], ssum.at[pl.ds(rbase, rpt)])
    plsc.subcore_barrier()

    base0 = (cid * NS + sid) * EPT

    def chunk(i, carry):
        base = base0 + i * CH
        pltpu.sync_copy(dst_hbm.at[pl.ds(base, CH)], idx)
        pltpu.sync_copy(m_hbm.at[pl.ds(base, CH)], rows)
        pltpu.sync_copy(rows, ssum.at[idx], add=True)
        return carry

    lax.fori_loop(0, NFULL, chunk, 0)

    tb = base0 + NFULL * CH
    pltpu.sync_copy(dst_hbm.at[pl.ds(tb, TAIL)], idx_t)
    pltpu.sync_copy(m_hbm.at[pl.ds(tb, TAIL)], rows.at[pl.ds(0, TAIL)])
    pltpu.sync_copy(rows.at[pl.ds(0, TAIL)], ssum.at[idx_t], add=True)

    plsc.subcore_barrier()
    pltpu.sync_copy(ssum.at[pl.ds(rbase, rpt)],
                    sums2_hbm.at[pl.ds(cid * N_PAD + rbase, rpt)])


def _stage_d(m, dst, zeros2):
    mesh = plsc.VectorSubcoreMesh(core_axis_name="c", subcore_axis_name="s",
                                  num_cores=NC, num_subcores=NS)
    f = pl.kernel(
        _stage_d_body,
        out_type=[
            jax.ShapeDtypeStruct((NC * N_PAD, H), jnp.float32),
        ],
        mesh=mesh,
        scratch_types=[
            pltpu.VMEM_SHARED((N_PAD, H), jnp.float32),
            pltpu.VMEM((CH,), jnp.int32),
            pltpu.VMEM((TAIL,), jnp.int32),
            pltpu.VMEM((CH, H), jnp.float32),
            pltpu.SemaphoreType.DMA,
        ],
    )
    return f(m, dst, zeros2)


def _stage_e_body(sums2_ref, cnt_ref, h_ref, mask_ref,
                  u1a_ref, u1b_ref, u1v_ref, u2m_ref, u2v_ref,
                  g_ref, b_ref, out_ref):
    s = sums2_ref[0] + sums2_ref[1]
    c = cnt_ref[...]
    m_i = s / jnp.maximum(c, 1.0)
    hv = h_ref[...]
    u = _silu(jnp.dot(hv, u1a_ref[...], preferred_element_type=jnp.float32)
              + jnp.dot(m_i, u1b_ref[...], preferred_element_type=jnp.float32)
              + u1v_ref[...])
    upd = jnp.dot(u, u2m_ref[...], preferred_element_type=jnp.float32) + u2v_ref[...]
    y = hv + upd
    mu = jnp.mean(y, axis=1, keepdims=True)
    var = jnp.mean((y - mu) ** 2, axis=1, keepdims=True)
    yn = (y - mu) / jnp.sqrt(var + 1e-5) * g_ref[...] + b_ref[...]
    out_ref[...] = jnp.where(mask_ref[...] > 0.5, yn, hv)


def _stage_e(sums2, cntr, h_pad, mask, u1a, u1b, u1v, u2m, u2v, g, b):
    blk = 1280
    grid = N_PAD // blk
    return pl.pallas_call(
        _stage_e_body,
        grid=(grid,),
        in_specs=[
            pl.BlockSpec((NC, blk, H), lambda i: (0, i, 0)),
            pl.BlockSpec((blk, 1), lambda i: (i, 0)),
            pl.BlockSpec((blk, H), lambda i: (i, 0)),
            pl.BlockSpec((blk, 1), lambda i: (i, 0)),
            pl.BlockSpec((H, H), lambda i: (0, 0)),
            pl.BlockSpec((H, H), lambda i: (0, 0)),
            pl.BlockSpec((1, H), lambda i: (0, 0)),
            pl.BlockSpec((H, H), lambda i: (0, 0)),
            pl.BlockSpec((1, H), lambda i: (0, 0)),
            pl.BlockSpec((1, H), lambda i: (0, 0)),
            pl.BlockSpec((1, H), lambda i: (0, 0)),
        ],
        out_specs=pl.BlockSpec((blk, H), lambda i: (i, 0)),
        out_shape=jax.ShapeDtypeStruct((N_PAD, H), jnp.float32),
    )(sums2, cntr, h_pad, mask, u1a, u1b, u1v, u2m, u2v, g, b)


@jax.jit
def kernel(h, pos, edge_index, edge_type, node_type, centers, emb,
           W1, b1, W2, b2, U1, u1, U2, u2, ln_g, ln_b):
    src = edge_index[0].astype(jnp.int32)
    dst = edge_index[1].astype(jnp.int32)

    w1a, w1b, w1c, w1d = W1[:H], W1[H:2 * H], W1[2 * H:3 * H], W1[3 * H:]
    step = centers[1] - centers[0]
    gamma = 1.0 / jnp.maximum(step * step, 1e-6)
    sg = jnp.sqrt(gamma)
    censg = (centers * sg).reshape(1, NUM_RBF)
    sg_arr = jnp.broadcast_to(sg, (1, NUM_RBF)).astype(jnp.float32)

    posflat = jnp.zeros((N, POS_W), jnp.float32).at[:, :3].set(pos).reshape(-1)
    etf = edge_type.astype(jnp.float32).reshape(E, 1)
    dstf = dst.astype(jnp.float32).reshape(E, 1)

    ha, hb, embc = _stage_a(h, w1a, w1b, emb, w1c, b1.reshape(1, H))
    preab, d2 = _stage_b(ha, hb, posflat, src, dst)
    m, cnt128 = _stage_c(preab, d2.reshape(E, 1), etf, dstf, embc, censg,
                         sg_arr, w1d, W2, b2.reshape(1, H))

    zeros2 = jnp.zeros((N_PAD, H), jnp.float32)
    (sums2,) = _stage_d(m, dst, zeros2)
    sums2 = sums2.reshape(NC, N_PAD, H)
    cntr = cnt128.reshape(-1)[:N_PAD].reshape(N_PAD, 1)

    h_pad = jnp.zeros((N_PAD, H), jnp.float32).at[:N].set(h)
    mask = jnp.zeros((N_PAD, 1), jnp.float32).at[:N, 0].set(
        (node_type == 1).astype(jnp.float32))

    u1a, u1b = U1[:H], U1[H:]
    out_pad = _stage_e(sums2, cntr, h_pad, mask, u1a, u1b,
                       u1.reshape(1, H), U2, u2.reshape(1, H),
                       ln_g.reshape(1, H), ln_b.reshape(1, H))
    return out_pad[:N]


# speedup vs baseline: 5.0225x; 1.1770x over previous
import jax
import jax.numpy as jnp
from jax import lax
from jax.experimental import pallas as pl
from jax.experimental.pallas import tpu as pltpu
from jax.experimental.pallas import tpu_sc as plsc

N = 10000
E = 320000
H = 128
NUM_RBF = 32
N_PAD = 10240

NC, NS = 2, 16
NW = NC * NS
EPT = E // NW
CH = 128
NFULL = EPT // CH
TAIL = EPT - NFULL * CH
POS_W = 4


def _silu(x):
    return x * jax.nn.sigmoid(x)


def _stage_a_body(h_ref, w1a_ref, w1b_ref, emb_ref, w1c_ref, b1_ref,
                  ha_ref, hb_ref, embc_ref):
    hv = h_ref[...]
    ha_ref[...] = jnp.dot(hv, w1a_ref[...], preferred_element_type=jnp.float32)
    hb_ref[...] = jnp.dot(hv, w1b_ref[...], preferred_element_type=jnp.float32)
    embc_ref[...] = (
        jnp.dot(emb_ref[...], w1c_ref[...], preferred_element_type=jnp.float32)
        + b1_ref[...])


def _stage_a(h, w1a, w1b, emb, w1c, b1):
    blk = 2000
    grid = N // blk
    return pl.pallas_call(
        _stage_a_body,
        grid=(grid,),
        in_specs=[
            pl.BlockSpec((blk, H), lambda i: (i, 0)),
            pl.BlockSpec((H, H), lambda i: (0, 0)),
            pl.BlockSpec((H, H), lambda i: (0, 0)),
            pl.BlockSpec((2, H), lambda i: (0, 0)),
            pl.BlockSpec((H, H), lambda i: (0, 0)),
            pl.BlockSpec((1, H), lambda i: (0, 0)),
        ],
        out_specs=[
            pl.BlockSpec((blk, H), lambda i: (i, 0)),
            pl.BlockSpec((blk, H), lambda i: (i, 0)),
            pl.BlockSpec((2, H), lambda i: (0, 0)),
        ],
        out_shape=[
            jax.ShapeDtypeStruct((N, H), jnp.float32),
            jax.ShapeDtypeStruct((N, H), jnp.float32),
            jax.ShapeDtypeStruct((2, H), jnp.float32),
        ],
    )(h, w1a, w1b, emb, w1c, b1)


def _stage_b_body(ha_hbm, hb_hbm, posflat_hbm, src_hbm, dst_hbm,
                  preab_hbm, d2_hbm,
                  idx_s0, idx_s1, idx_d0, idx_d1, idx_s_t, idx_d_t,
                  rows0, rows1, d2b0, d2b1, posv,
                  sga0, sga1, sgb0, sgb1, sst0, sst1):
    wid = lax.axis_index("c") * NS + lax.axis_index("s")
    base0 = wid * EPT
    idx_s = (idx_s0, idx_s1)
    idx_d = (idx_d0, idx_d1)
    rows = (rows0, rows1)
    d2b = (d2b0, d2b1)
    sga = (sga0, sga1)
    sgb = (sgb0, sgb1)
    sst = (sst0, sst1)

    pltpu.sync_copy(posflat_hbm, posv)

    def dist2_chunk(si, di, out, n_groups):
        for g in range(n_groups):
            s16 = si[pl.ds(g * 16, 16)]
            d16 = di[pl.ds(g * 16, 16)]
            sb = s16 * POS_W
            db = d16 * POS_W
            acc = jnp.zeros((16,), jnp.float32)
            for k in range(3):
                a = plsc.load_gather(posv, [sb + k])
                bb = plsc.load_gather(posv, [db + k])
                r = a - bb
                acc = acc + r * r
            out[pl.ds(g * 16, 16)] = acc

    def issue_store(j, slot):
        base = base0 + j * CH
        pltpu.async_copy(rows[slot], preab_hbm.at[pl.ds(base, CH)], sst[slot])
        pltpu.async_copy(d2b[slot], d2_hbm.at[pl.ds(base, CH)], sst[slot])

    def wait_store(j, slot):
        base = base0 + j * CH
        pltpu.make_async_copy(rows[slot], preab_hbm.at[pl.ds(base, CH)],
                              sst[slot]).wait()
        pltpu.make_async_copy(d2b[slot], d2_hbm.at[pl.ds(base, CH)],
                              sst[slot]).wait()

    pltpu.sync_copy(src_hbm.at[pl.ds(base0, CH)], idx_s[0])
    pltpu.sync_copy(dst_hbm.at[pl.ds(base0, CH)], idx_d[0])
    pltpu.async_copy(ha_hbm.at[idx_s[0]], rows[0], sga[0])

    def pair(g, carry):
        for b in (0, 1):
            i = g * 2 + b
            nb = 1 - b
            if b == 0:
                pltpu.sync_copy(src_hbm.at[pl.ds(base0 + (i + 1) * CH, CH)],
                                idx_s[nb])
            else:
                @pl.when(g < NFULL // 2 - 1)
                def _():
                    pltpu.sync_copy(
                        src_hbm.at[pl.ds(base0 + (i + 1) * CH, CH)], idx_s[nb])
            pltpu.make_async_copy(ha_hbm.at[idx_s[b]], rows[b], sga[b]).wait()
            pltpu.async_copy(hb_hbm.at[idx_d[b]], rows[b], sgb[b], add=True)
            dist2_chunk(idx_s[b], idx_d[b], d2b[b], CH // 16)
            if b == 0:
                @pl.when(g > 0)
                def _():
                    pltpu.make_async_copy(hb_hbm.at[idx_d[nb]], rows[nb],
                                          sgb[nb]).wait()
                    issue_store(i - 1, nb)
                    pltpu.sync_copy(
                        dst_hbm.at[pl.ds(base0 + (i + 1) * CH, CH)], idx_d[nb])
                    wait_store(i - 1, nb)

                @pl.when(g == 0)
                def _():
                    pltpu.sync_copy(
                        dst_hbm.at[pl.ds(base0 + (i + 1) * CH, CH)], idx_d[nb])
                pltpu.async_copy(ha_hbm.at[idx_s[nb]], rows[nb], sga[nb])
            else:
                @pl.when(g < NFULL // 2 - 1)
                def _():
                    pltpu.make_async_copy(hb_hbm.at[idx_d[nb]], rows[nb],
                                          sgb[nb]).wait()
                    issue_store(i - 1, nb)
                    pltpu.sync_copy(
                        dst_hbm.at[pl.ds(base0 + (i + 1) * CH, CH)], idx_d[nb])
                    wait_store(i - 1, nb)
                    pltpu.async_copy(ha_hbm.at[idx_s[nb]], rows[nb], sga[nb])
        return carry

    lax.fori_loop(0, NFULL // 2, pair, 0)

    last = NFULL - 1
    pltpu.make_async_copy(hb_hbm.at[idx_d[0]], rows[0], sgb[0]).wait()
    issue_store(last - 1, 0)
    wait_store(last - 1, 0)
    pltpu.make_async_copy(hb_hbm.at[idx_d[1]], rows[1], sgb[1]).wait()
    issue_store(last, 1)
    wait_store(last, 1)

    tb = base0 + NFULL * CH
    pltpu.sync_copy(src_hbm.at[pl.ds(tb, TAIL)], idx_s_t)
    pltpu.sync_copy(dst_hbm.at[pl.ds(tb, TAIL)], idx_d_t)
    pltpu.async_copy(ha_hbm.at[idx_s_t], rows0.at[pl.ds(0, TAIL)], sga0).wait()
    pltpu.async_copy(hb_hbm.at[idx_d_t], rows0.at[pl.ds(0, TAIL)], sga0,
                     add=True).wait()
    dist2_chunk(idx_s_t, idx_d_t, d2b0, TAIL // 16)
    pltpu.sync_copy(rows0.at[pl.ds(0, TAIL)], preab_hbm.at[pl.ds(tb, TAIL)])
    pltpu.sync_copy(d2b0.at[pl.ds(0, TAIL)], d2_hbm.at[pl.ds(tb, TAIL)])


def _stage_b(ha, hb, posflat, src, dst):
    mesh = plsc.VectorSubcoreMesh(core_axis_name="c", subcore_axis_name="s",
                                  num_cores=NC, num_subcores=NS)
    f = pl.kernel(
        _stage_b_body,
        out_type=[
            jax.ShapeDtypeStruct((E, H), jnp.float32),
            jax.ShapeDtypeStruct((E,), jnp.float32),
        ],
        mesh=mesh,
        scratch_types=[
            pltpu.VMEM((CH,), jnp.int32),
            pltpu.VMEM((CH,), jnp.int32),
            pltpu.VMEM((CH,), jnp.int32),
            pltpu.VMEM((CH,), jnp.int32),
            pltpu.VMEM((TAIL,), jnp.int32),
            pltpu.VMEM((TAIL,), jnp.int32),
            pltpu.VMEM((CH, H), jnp.float32),
            pltpu.VMEM((CH, H), jnp.float32),
            pltpu.VMEM((CH,), jnp.float32),
            pltpu.VMEM((CH,), jnp.float32),
            pltpu.VMEM((N * POS_W,), jnp.float32),
            pltpu.SemaphoreType.DMA,
            pltpu.SemaphoreType.DMA,
            pltpu.SemaphoreType.DMA,
            pltpu.SemaphoreType.DMA,
            pltpu.SemaphoreType.DMA,
            pltpu.SemaphoreType.DMA,
        ],
        compiler_params=pltpu.CompilerParams(needs_layout_passes=False),
    )
    return f(ha, hb, posflat, src, dst)


def _stage_c_body(preab_ref, d2_ref, et_ref, dst_ref, embc_ref, censg_ref,
                  sg_ref, w1d_ref, w2_ref, b2_ref, m_ref, cnt_ref):
    i = pl.program_id(0)

    d2 = d2_ref[...]
    dist = jnp.sqrt(d2)
    diff = dist * sg_ref[...] - censg_ref[...]
    radial = jnp.exp(-diff * diff)
    ec = embc_ref[...]
    et = et_ref[...]
    ek = ec[0:1, :] + et * (ec[1:2, :] - ec[0:1, :])
    pre = (preab_ref[...] + ek
           + jnp.dot(radial, w1d_ref[...], preferred_element_type=jnp.float32))
    x = _silu(pre)
    xm = jnp.dot(x, w2_ref[...], preferred_element_type=jnp.float32) + b2_ref[...]
    m_ref[...] = _silu(xm)

    blk = dst_ref.shape[0]
    dstf = dst_ref[...]
    qf = jnp.floor(dstf * (1.0 / 128.0))
    rf = dstf - qf * 128.0
    lane = lax.broadcasted_iota(jnp.int32, (blk, 128), 1).astype(jnp.float32)
    oh_r = jnp.where(rf == lane, 1.0, 0.0)
    oh_q = jnp.where(qf == lane, 1.0, 0.0)
    part = lax.dot_general(oh_q, oh_r, (((0,), (0,)), ((), ())),
                           preferred_element_type=jnp.float32)

    @pl.when(i == 0)
    def _():
        cnt_ref[...] = jnp.zeros_like(cnt_ref)

    cnt_ref[...] += part


def _stage_c(preab, d2, etf, dstf, embc, censg, sg, w1d, w2, b2):
    blk = 2000
    grid = E // blk
    return pl.pallas_call(
        _stage_c_body,
        grid=(grid,),
        in_specs=[
            pl.BlockSpec((blk, H), lambda i: (i, 0)),
            pl.BlockSpec((blk, 1), lambda i: (i, 0)),
            pl.BlockSpec((blk, 1), lambda i: (i, 0)),
            pl.BlockSpec((blk, 1), lambda i: (i, 0)),
            pl.BlockSpec((2, H), lambda i: (0, 0)),
            pl.BlockSpec((1, NUM_RBF), lambda i: (0, 0)),
            pl.BlockSpec((1, NUM_RBF), lambda i: (0, 0)),
            pl.BlockSpec((NUM_RBF, H), lambda i: (0, 0)),
            pl.BlockSpec((H, H), lambda i: (0, 0)),
            pl.BlockSpec((1, H), lambda i: (0, 0)),
        ],
        out_specs=[
            pl.BlockSpec((blk, H), lambda i: (i, 0)),
            pl.BlockSpec((128, 128), lambda i: (0, 0)),
        ],
        out_shape=[
            jax.ShapeDtypeStruct((E, H), jnp.float32),
            jax.ShapeDtypeStruct((128, 128), jnp.float32),
        ],
    )(preab, d2, etf, dstf, embc, censg, sg, w1d, w2, b2)


def _stage_d_body(m_hbm, dst_hbm, zeros2_hbm, sums2_hbm,
                  ssum, idx0, idx1, idx_t, rows0, rows1,
                  sml0, sml1, ssc0, ssc1):
    cid = lax.axis_index("c")
    sid = lax.axis_index("s")
    rpt = N_PAD // NS
    rbase = sid * rpt
    idx = (idx0, idx1)
    rows = (rows0, rows1)
    sml = (sml0, sml1)
    ssc = (ssc0, ssc1)

    pltpu.sync_copy(zeros2_hbm.at[pl.ds(rbase, rpt)], ssum.at[pl.ds(rbase, rpt)])
    plsc.subcore_barrier()

    base0 = (cid * NS + sid) * EPT

    pltpu.sync_copy(dst_hbm.at[pl.ds(base0, CH)], idx[0])
    pltpu.async_copy(m_hbm.at[pl.ds(base0, CH)], rows[0], sml[0])

    def pair(g, carry):
        for b in (0, 1):
            i = g * 2 + b
            nb = 1 - b
            base = base0 + i * CH
            pltpu.make_async_copy(m_hbm.at[pl.ds(base, CH)], rows[b],
                                  sml[b]).wait()
            pltpu.async_copy(rows[b], ssum.at[idx[b]], ssc[b], add=True)

            def advance():
                nbase = base0 + (i + 1) * CH
                pltpu.sync_copy(dst_hbm.at[pl.ds(nbase, CH)], idx[nb])
                pltpu.async_copy(m_hbm.at[pl.ds(nbase, CH)], rows[nb], sml[nb])

            if b == 0:
                @pl.when(g > 0)
                def _():
                    pltpu.make_async_copy(rows[nb], ssum.at[idx[nb]],
                                          ssc[nb]).wait()
                advance()
            else:
                @pl.when(g < NFULL // 2 - 1)
                def _():
                    pltpu.make_async_copy(rows[nb], ssum.at[idx[nb]],
                                          ssc[nb]).wait()
                    advance()
        return carry

    lax.fori_loop(0, NFULL // 2, pair, 0)

    pltpu.make_async_copy(rows[0], ssum.at[idx[0]], ssc[0]).wait()
    pltpu.make_async_copy(rows[1], ssum.at[idx[1]], ssc[1]).wait()

    tb = base0 + NFULL * CH
    pltpu.sync_copy(dst_hbm.at[pl.ds(tb, TAIL)], idx_t)
    pltpu.sync_copy(m_hbm.at[pl.ds(tb, TAIL)], rows0.at[pl.ds(0, TAIL)])
    pltpu.sync_copy(rows0.at[pl.ds(0, TAIL)], ssum.at[idx_t], add=True)

    plsc.subcore_barrier()
    pltpu.sync_copy(ssum.at[pl.ds(rbase, rpt)],
                    sums2_hbm.at[pl.ds(cid * N_PAD + rbase, rpt)])


def _stage_d(m, dst, zeros2):
    mesh = plsc.VectorSubcoreMesh(core_axis_name="c", subcore_axis_name="s",
                                  num_cores=NC, num_subcores=NS)
    f = pl.kernel(
        _stage_d_body,
        out_type=[
            jax.ShapeDtypeStruct((NC * N_PAD, H), jnp.float32),
        ],
        mesh=mesh,
        scratch_types=[
            pltpu.VMEM_SHARED((N_PAD, H), jnp.float32),
            pltpu.VMEM((CH,), jnp.int32),
            pltpu.VMEM((CH,), jnp.int32),
            pltpu.VMEM((TAIL,), jnp.int32),
            pltpu.VMEM((CH, H), jnp.float32),
            pltpu.VMEM((CH, H), jnp.float32),
            pltpu.SemaphoreType.DMA,
            pltpu.SemaphoreType.DMA,
            pltpu.SemaphoreType.DMA,
            pltpu.SemaphoreType.DMA,
        ],
    )
    return f(m, dst, zeros2)


def _stage_e_body(sums2_ref, cnt_ref, h_ref, mask_ref,
                  u1a_ref, u1b_ref, u1v_ref, u2m_ref, u2v_ref,
                  g_ref, b_ref, out_ref):
    s = sums2_ref[0] + sums2_ref[1]
    c = cnt_ref[...]
    m_i = s / jnp.maximum(c, 1.0)
    hv = h_ref[...]
    u = _silu(jnp.dot(hv, u1a_ref[...], preferred_element_type=jnp.float32)
              + jnp.dot(m_i, u1b_ref[...], preferred_element_type=jnp.float32)
              + u1v_ref[...])
    upd = jnp.dot(u, u2m_ref[...], preferred_element_type=jnp.float32) + u2v_ref[...]
    y = hv + upd
    mu = jnp.mean(y, axis=1, keepdims=True)
    var = jnp.mean((y - mu) ** 2, axis=1, keepdims=True)
    yn = (y - mu) / jnp.sqrt(var + 1e-5) * g_ref[...] + b_ref[...]
    out_ref[...] = jnp.where(mask_ref[...] > 0.5, yn, hv)


def _stage_e(sums2, cntr, h_pad, mask, u1a, u1b, u1v, u2m, u2v, g, b):
    blk = 1280
    grid = N_PAD // blk
    return pl.pallas_call(
        _stage_e_body,
        grid=(grid,),
        in_specs=[
            pl.BlockSpec((NC, blk, H), lambda i: (0, i, 0)),
            pl.BlockSpec((blk, 1), lambda i: (i, 0)),
            pl.BlockSpec((blk, H), lambda i: (i, 0)),
            pl.BlockSpec((blk, 1), lambda i: (i, 0)),
            pl.BlockSpec((H, H), lambda i: (0, 0)),
            pl.BlockSpec((H, H), lambda i: (0, 0)),
            pl.BlockSpec((1, H), lambda i: (0, 0)),
            pl.BlockSpec((H, H), lambda i: (0, 0)),
            pl.BlockSpec((1, H), lambda i: (0, 0)),
            pl.BlockSpec((1, H), lambda i: (0, 0)),
            pl.BlockSpec((1, H), lambda i: (0, 0)),
        ],
        out_specs=pl.BlockSpec((blk, H), lambda i: (i, 0)),
        out_shape=jax.ShapeDtypeStruct((N_PAD, H), jnp.float32),
    )(sums2, cntr, h_pad, mask, u1a, u1b, u1v, u2m, u2v, g, b)


@jax.jit
def kernel(h, pos, edge_index, edge_type, node_type, centers, emb,
           W1, b1, W2, b2, U1, u1, U2, u2, ln_g, ln_b):
    src = edge_index[0].astype(jnp.int32)
    dst = edge_index[1].astype(jnp.int32)

    w1a, w1b, w1c, w1d = W1[:H], W1[H:2 * H], W1[2 * H:3 * H], W1[3 * H:]
    step = centers[1] - centers[0]
    gamma = 1.0 / jnp.maximum(step * step, 1e-6)
    sg = jnp.sqrt(gamma)
    censg = (centers * sg).reshape(1, NUM_RBF)
    sg_arr = jnp.broadcast_to(sg, (1, NUM_RBF)).astype(jnp.float32)

    posflat = jnp.zeros((N, POS_W), jnp.float32).at[:, :3].set(pos).reshape(-1)
    etf = edge_type.astype(jnp.float32).reshape(E, 1)
    dstf = dst.astype(jnp.float32).reshape(E, 1)

    ha, hb, embc = _stage_a(h, w1a, w1b, emb, w1c, b1.reshape(1, H))
    preab, d2 = _stage_b(ha, hb, posflat, src, dst)
    m, cnt128 = _stage_c(preab, d2.reshape(E, 1), etf, dstf, embc, censg,
                         sg_arr, w1d, W2, b2.reshape(1, H))

    zeros2 = jnp.zeros((N_PAD, H), jnp.float32)
    (sums2,) = _stage_d(m, dst, zeros2)
    sums2 = sums2.reshape(NC, N_PAD, H)
    cntr = cnt128.reshape(-1)[:N_PAD].reshape(N_PAD, 1)

    h_pad = jnp.zeros((N_PAD, H), jnp.float32).at[:N].set(h)
    mask = jnp.zeros((N_PAD, 1), jnp.float32).at[:N, 0].set(
        (node_type == 1).astype(jnp.float32))

    u1a, u1b = U1[:H], U1[H:]
    out_pad = _stage_e(sums2, cntr, h_pad, mask, u1a, u1b,
                       u1.reshape(1, H), U2, u2.reshape(1, H),
                       ln_g.reshape(1, H), ln_b.reshape(1, H))
    return out_pad[:N]
